# Initial kernel scaffold; baseline (speedup 1.0000x reference)
import functools

import jax
import jax.numpy as jnp
from jax import lax
from jax.experimental import pallas as pl
from jax.experimental.pallas import tpu as pltpu
from jax.experimental.pallas import tpu_sc as plsc

VOCAB = 1000000
EMBED_DIM = 32
BATCH = 16384
HIST = 50

NC = 2
NS = 16
NW = NC * NS
TOTAL = BATCH * HIST
PER_W = TOTAL // NW
CHUNK = 1024
K = CHUNK // 128
NCHUNK = PER_W // CHUNK


def _mesh():
    return plsc.VectorSubcoreMesh(core_axis_name="c", subcore_axis_name="s")


@functools.partial(
    pl.kernel,
    mesh=_mesh(),
    out_type=jax.ShapeDtypeStruct((TOTAL, EMBED_DIM), jnp.float32),
    scratch_types=[
        pltpu.VMEM((K, 128), jnp.int32),
        pltpu.VMEM((CHUNK, EMBED_DIM), jnp.float32),
        pltpu.SemaphoreType.DMA,
    ],
)
def _gather_kernel(idx_hbm, table_hbm, out_hbm, idx_v, rows_v, gsem):
    wid = lax.axis_index("s") * NC + lax.axis_index("c")

    def chunk_body(c, carry):
        pltpu.sync_copy(idx_hbm.at[wid, c], idx_v)
        copies = [
            pltpu.async_copy(
                table_hbm.at[idx_v.at[j]],
                rows_v.at[pl.ds(j * 128, 128)],
                gsem,
            )
            for j in range(K)
        ]
        for cp in copies:
            cp.wait()
        pltpu.sync_copy(
            rows_v, out_hbm.at[pl.ds(wid * PER_W + c * CHUNK, CHUNK)]
        )
        return carry

    lax.fori_loop(0, NCHUNK, chunk_body, 0)


def kernel(x, table):
    idx = x.reshape(NW, NCHUNK, K, 128).astype(jnp.int32)
    out = _gather_kernel(idx, table)
    return out.reshape(BATCH, HIST, EMBED_DIM)


# speedup vs baseline: 1.0934x; 1.0934x over previous
import functools

import jax
import jax.numpy as jnp
from jax import lax
from jax.experimental import pallas as pl
from jax.experimental.pallas import tpu as pltpu
from jax.experimental.pallas import tpu_sc as plsc

VOCAB = 1000000
EMBED_DIM = 32
BATCH = 16384
HIST = 50

NC = 2
NS = 16
NW = NC * NS
TOTAL = BATCH * HIST
PER_W = TOTAL // NW
CHUNK = 1024
K = CHUNK // 128
NCHUNK = PER_W // CHUNK


def _mesh():
    return plsc.VectorSubcoreMesh(core_axis_name="c", subcore_axis_name="s")


@functools.partial(
    pl.kernel,
    mesh=_mesh(),
    compiler_params=pltpu.CompilerParams(use_tc_tiling_on_sc=False),
    out_type=jax.ShapeDtypeStruct((TOTAL, EMBED_DIM), jnp.float32),
    scratch_types=[
        pltpu.VMEM((K, 128), jnp.int32),
        pltpu.VMEM((CHUNK, EMBED_DIM), jnp.float32),
        pltpu.SemaphoreType.DMA,
    ],
)
def _gather_kernel(idx_hbm, table_hbm, out_hbm, idx_v, rows_v, gsem):
    wid = lax.axis_index("s") * NC + lax.axis_index("c")

    def chunk_body(c, carry):
        pltpu.sync_copy(idx_hbm.at[wid, c], idx_v)
        copies = [
            pltpu.async_copy(
                table_hbm.at[idx_v.at[j]],
                rows_v.at[pl.ds(j * 128, 128)],
                gsem,
            )
            for j in range(K)
        ]
        for cp in copies:
            cp.wait()
        pltpu.sync_copy(
            rows_v, out_hbm.at[pl.ds(wid * PER_W + c * CHUNK, CHUNK)]
        )
        return carry

    lax.fori_loop(0, NCHUNK, chunk_body, 0)


def kernel(x, table):
    idx = x.reshape(NW, NCHUNK, K, 128).astype(jnp.int32)
    out = _gather_kernel(idx, table)
    return out.reshape(BATCH, HIST, EMBED_DIM)


# speedup vs baseline: 1.1130x; 1.0180x over previous
import functools

import jax
import jax.numpy as jnp
from jax import lax
from jax.experimental import pallas as pl
from jax.experimental.pallas import tpu as pltpu
from jax.experimental.pallas import tpu_sc as plsc

VOCAB = 1000000
EMBED_DIM = 32
BATCH = 16384
HIST = 50

NC = 2
NS = 16
NW = NC * NS
TOTAL = BATCH * HIST
PER_W = TOTAL // NW
CHUNK = 640
K = CHUNK // 128
NCHUNK = PER_W // CHUNK
NBUF = 4
NG = NCHUNK // NBUF


def _mesh():
    return plsc.VectorSubcoreMesh(core_axis_name="c", subcore_axis_name="s")


@functools.partial(
    pl.kernel,
    mesh=_mesh(),
    compiler_params=pltpu.CompilerParams(use_tc_tiling_on_sc=False),
    out_type=jax.ShapeDtypeStruct((TOTAL, EMBED_DIM), jnp.float32),
    scratch_types=[
        pltpu.VMEM((NBUF, K, 128), jnp.int32),
        pltpu.VMEM((NBUF, CHUNK, EMBED_DIM), jnp.float32),
        [pltpu.SemaphoreType.DMA] * NBUF,
        [pltpu.SemaphoreType.DMA] * NBUF,
    ],
)
def _gather_kernel(idx_hbm, table_hbm, out_hbm, idx_v, rows_v, gsems, osems):
    wid = lax.axis_index("s") * NC + lax.axis_index("c")
    base = wid * PER_W

    def fire(b, c):
        pltpu.sync_copy(idx_hbm.at[wid, c], idx_v.at[b])
        for j in range(K):
            pltpu.async_copy(
                table_hbm.at[idx_v.at[b, j]],
                rows_v.at[b, pl.ds(j * 128, 128)],
                gsems[b],
            )

    def drain_gathers(b):
        pltpu.make_async_copy(
            table_hbm.at[pl.ds(0, CHUNK)], rows_v.at[b], gsems[b]
        ).wait()

    def store(b, c):
        pltpu.async_copy(
            rows_v.at[b], out_hbm.at[pl.ds(base + c * CHUNK, CHUNK)], osems[b]
        )

    def drain_store(b):
        pltpu.make_async_copy(
            rows_v.at[b], out_hbm.at[pl.ds(0, CHUNK)], osems[b]
        ).wait()

    fire(0, 0)
    fire(1, 1)

    for off in range(NBUF):
        c = off
        bf = (off + 2) % NBUF
        if off >= 2:
            drain_store(bf)
        fire(bf, c + 2)
        drain_gathers(off)
        store(off, c)

    def outer(g, carry):
        c0 = g * NBUF
        for off in range(NBUF):
            c = c0 + off
            bf = (off + 2) % NBUF
            drain_store(bf)
            fire(bf, c + 2)
            drain_gathers(off)
            store(off, c)
        return carry

    lax.fori_loop(1, NG - 1, outer, 0)

    c0 = (NG - 1) * NBUF
    for off in range(NBUF):
        c = c0 + off
        bf = (off + 2) % NBUF
        if off < 2:
            drain_store(bf)
            fire(bf, c + 2)
        drain_gathers(off)
        store(off, c)
    for b in range(NBUF):
        drain_store(b)


def kernel(x, table):
    idx = x.reshape(NW, NCHUNK, K, 128).astype(jnp.int32)
    out = _gather_kernel(idx, table)
    return out.reshape(BATCH, HIST, EMBED_DIM)


# speedup vs baseline: 1.8639x; 1.6746x over previous
import functools

import jax
import jax.numpy as jnp
from jax import lax
from jax.experimental import pallas as pl
from jax.experimental.pallas import tpu as pltpu
from jax.experimental.pallas import tpu_sc as plsc

VOCAB = 1000000
EMBED_DIM = 32
BATCH = 16384
HIST = 50

NC = 2
NS = 16
NW = NC * NS
NUNIT = HIST * (BATCH // 128)
PER_W = NUNIT // NW


def _mesh():
    return plsc.VectorSubcoreMesh(core_axis_name="c", subcore_axis_name="s")


@functools.partial(
    pl.kernel,
    mesh=_mesh(),
    compiler_params=pltpu.CompilerParams(
        use_tc_tiling_on_sc=False, needs_layout_passes=False
    ),
    out_type=jax.ShapeDtypeStruct((HIST * EMBED_DIM * BATCH,), jnp.float32),
    scratch_types=[
        [pltpu.VMEM((128,), jnp.int32)] * 2,
        [pltpu.VMEM((128, EMBED_DIM), jnp.float32)] * 2,
        [pltpu.VMEM((4096,), jnp.float32)] * 2,
        [pltpu.SemaphoreType.DMA] * 2,
        [pltpu.SemaphoreType.DMA] * 2,
        [pltpu.SemaphoreType.DMA] * 2,
    ],
)
def _gather_kernel(tab_hbm, idx_hbm, out_hbm, idx_v, rows_v, tbuf, isems,
                   gsems, osems):
    wid = lax.axis_index("s") * NC + lax.axis_index("c")
    i16 = jnp.arange(16, dtype=jnp.int32)
    a_lo = (i16 // 8) * 1024 + (i16 % 8) * 128
    a_hi = a_lo + 2048

    def unit_id(i):
        return wid + NW * i

    def fire_idx(b, i):
        pltpu.async_copy(idx_hbm.at[unit_id(i)], idx_v[b], isems[b])

    def wait_idx(b):
        pltpu.make_async_copy(idx_hbm.at[0], idx_v[b], isems[b]).wait()

    def fire_gather(b):
        pltpu.async_copy(tab_hbm.at[idx_v[b]], rows_v[b], gsems[b])

    def drain_gather(b):
        pltpu.make_async_copy(
            tab_hbm.at[pl.ds(0, 128)], rows_v[b], gsems[b]
        ).wait()

    def transpose(b):
        def tbody(k, carry):
            for j in range(4):
                cc = k * 4 + j
                lo = rows_v[b][cc, pl.ds(0, 16)]
                hi = rows_v[b][cc, pl.ds(16, 16)]
                plsc.store_scatter(tbuf[b], [a_lo + cc], lo)
                plsc.store_scatter(tbuf[b], [a_hi + cc], hi)
            return carry

        lax.fori_loop(0, 32, tbody, 0)

    def store(b, i):
        u = unit_id(i)
        h = u // 128
        cc = lax.rem(u, 128)
        for r in range(4):
            off = ((h * 4 + r) * 128 + cc) * 1024
            pltpu.async_copy(
                tbuf[b].at[pl.ds(r * 1024, 1024)],
                out_hbm.at[pl.ds(off, 1024)],
                osems[b],
            )

    def drain_store(b):
        for _ in range(4):
            pltpu.make_async_copy(
                tbuf[b].at[pl.ds(0, 1024)], out_hbm.at[pl.ds(0, 1024)],
                osems[b],
            ).wait()

    fire_idx(0, 0)
    wait_idx(0)
    fire_gather(0)
    fire_idx(1, 1)

    def body(k, carry):
        for p in range(2):
            i = 2 * k + p
            nxt = 1 - p

            drain_gather(p)

            @pl.when(i + 1 < PER_W)
            def _():
                wait_idx(nxt)
                fire_gather(nxt)

            @pl.when(i + 2 < PER_W)
            def _():
                fire_idx(p, i + 2)

            @pl.when(i >= 2)
            def _():
                drain_store(p)

            transpose(p)
            store(p, i)
        return carry

    lax.fori_loop(0, PER_W // 2, body, 0)
    drain_store(0)
    drain_store(1)


def kernel(x, table):
    idx = x.T.reshape(NUNIT, 128).astype(jnp.int32)
    out_flat = _gather_kernel(table, idx)
    out5 = out_flat.reshape(HIST, 4, 128, 8, 128)
    return jnp.transpose(out5, (2, 4, 0, 1, 3)).reshape(BATCH, HIST, EMBED_DIM)
